# baseline (speedup 1.0000x reference)
import jax
import jax.numpy as jnp
from jax.experimental import pallas as pl

_N_F = 26
_N_PAIRS = (_N_F * (_N_F - 1)) // 2


def _body(x_ref, o_ref):
    x = x_ref[...]
    off = 0
    for i in range(_N_F - 1):
        w = _N_F - 1 - i
        seg = x[:, i + 1:, :] * x[:, i:i + 1, :]
        o_ref[:, off:off + w] = jnp.sum(seg, axis=-1)
        off += w


def kernel(inputs):
    n = inputs.shape[0]
    B = 256
    return pl.pallas_call(
        _body,
        grid=(n // B,),
        in_specs=[pl.BlockSpec((B, _N_F, 16), lambda i: (i, 0, 0))],
        out_specs=pl.BlockSpec((B, _N_PAIRS), lambda i: (i, 0)),
        out_shape=jax.ShapeDtypeStruct((n, _N_PAIRS), jnp.float32),
    )(inputs)


# speedup vs baseline: 1.6778x; 1.6778x over previous
import jax
import jax.numpy as jnp
from jax import lax
from jax.experimental import pallas as pl
from jax.experimental.pallas import tpu as pltpu
from jax.experimental.pallas import tpu_sc as plsc

_NF = 26
_D = 16
_ROW = _NF * _D
_NP = (_NF * (_NF - 1)) // 2
_NC = 2
_NS = 16
_NW = _NC * _NS
_L = 16
_BS = 5
_C = 128


def _pair_k(i, j):
    return 25 * i - (i * (i - 1)) // 2 + (j - i - 1)


def _make_body(n):
    rows_per = n // _NW
    n_chunks = rows_per // _C
    n_groups = _C // _L

    def body(x_hbm, o_hbm, x_v, o_v):
        wid = lax.axis_index("s") * _NC + lax.axis_index("c")
        base = wid * rows_per
        lane = lax.iota(jnp.int32, _L)

        def chunk_body(c, carry):
            row0 = base + c * _C
            pltpu.sync_copy(x_hbm.at[pl.ds(row0 * _ROW, _C * _ROW)], x_v)

            def group_body(g, carry2):
                idx_x = (g * _L + lane) * _ROW
                idx_o = (g * _L + lane) * _NP

                def ld(f, d):
                    return plsc.load_gather(x_v, [idx_x + (f * _D + d)])

                def st(k, v):
                    plsc.store_scatter(o_v, [idx_o + k], v)

                def offdiag(t, c3):
                    bi = ((t >= 4).astype(jnp.int32)
                          + (t >= 7).astype(jnp.int32)
                          + (t >= 9).astype(jnp.int32))
                    bj = t - (4 * bi - (bi * (bi - 1)) // 2) + bi + 1
                    ib = _BS * bi
                    jb = _BS * bj
                    acc = [[jnp.zeros((_L,), jnp.float32)
                            for _ in range(_BS)] for _ in range(_BS)]
                    for d in range(_D):
                        pa = [ld(ib + a, d) for a in range(_BS)]
                        qa = [ld(jb + b, d) for b in range(_BS)]
                        for a in range(_BS):
                            for b in range(_BS):
                                acc[a][b] = acc[a][b] + pa[a] * qa[b]
                    for a in range(_BS):
                        for b in range(_BS):
                            st(_pair_k(ib + a, jb + b), acc[a][b])
                    return c3

                lax.fori_loop(0, 10, offdiag, 0)

                def diag(t, c3):
                    ib = _BS * t
                    acc = {}
                    for a in range(_BS):
                        for b in range(a + 1, _BS):
                            acc[(a, b)] = jnp.zeros((_L,), jnp.float32)
                    for d in range(_D):
                        pa = [ld(ib + a, d) for a in range(_BS)]
                        for a in range(_BS):
                            for b in range(a + 1, _BS):
                                acc[(a, b)] = acc[(a, b)] + pa[a] * pa[b]
                    for a in range(_BS):
                        for b in range(a + 1, _BS):
                            st(_pair_k(ib + a, ib + b), acc[(a, b)])
                    return c3

                lax.fori_loop(0, 5, diag, 0)

                last = [ld(_NF - 1, d) for d in range(_D)]

                def col(t, c3):
                    ib = _BS * t
                    acc = [jnp.zeros((_L,), jnp.float32)
                           for _ in range(_BS)]
                    for d in range(_D):
                        pa = [ld(ib + a, d) for a in range(_BS)]
                        for a in range(_BS):
                            acc[a] = acc[a] + pa[a] * last[d]
                    for a in range(_BS):
                        st(_pair_k(ib + a, _NF - 1), acc[a])
                    return c3

                lax.fori_loop(0, 5, col, 0)
                return carry2

            lax.fori_loop(0, n_groups, group_body, 0)
            pltpu.sync_copy(o_v, o_hbm.at[pl.ds(row0 * _NP, _C * _NP)])
            return carry

        lax.fori_loop(0, n_chunks, chunk_body, 0)

    return body


def kernel(inputs):
    n = inputs.shape[0]
    mesh = plsc.VectorSubcoreMesh(core_axis_name="c", subcore_axis_name="s",
                                  num_cores=_NC, num_subcores=_NS)
    f = pl.kernel(
        _make_body(n),
        out_type=jax.ShapeDtypeStruct((n * _NP,), jnp.float32),
        mesh=mesh,
        scratch_types=[pltpu.VMEM((_C * _ROW,), jnp.float32),
                       pltpu.VMEM((_C * _NP,), jnp.float32)],
        compiler_params=pltpu.CompilerParams(use_tc_tiling_on_sc=False,
                                             needs_layout_passes=False),
    )
    flat = f(inputs.reshape(n * _ROW))
    return flat.reshape(n, _NP)


# speedup vs baseline: 3.1632x; 1.8853x over previous
import jax
import jax.numpy as jnp
from jax import lax
from jax.experimental import pallas as pl
from jax.experimental.pallas import tpu as pltpu
from jax.experimental.pallas import tpu_sc as plsc

_NF = 26
_D = 16
_ROW = _NF * _D
_NP = (_NF * (_NF - 1)) // 2
_NC = 2
_NS = 16
_NW = _NC * _NS
_L = 16
_BS = 5
_C = 128
_TS = 17


def _pair_k(i, j):
    return 25 * i - (i * (i - 1)) // 2 + (j - i - 1)


def _make_body(n):
    rows_per = n // _NW
    n_chunks = rows_per // _C
    n_groups = _C // _L

    def body(x_hbm, o_hbm, x_v, o_v, xt_v):
        wid = lax.axis_index("s") * _NC + lax.axis_index("c")
        base = wid * rows_per
        lane = lax.iota(jnp.int32, _L)
        lane17 = lane * _TS

        def chunk_body(c, carry):
            row0 = base + c * _C
            pltpu.sync_copy(x_hbm.at[pl.ds(row0 * _ROW, _C * _ROW)], x_v)

            def group_body(g, carry2):
                idx_o = (g * _L + lane) * _NP

                def repack(b, c3):
                    src = (g * _L + b) * _ROW
                    dst = lane17 + b
                    for r in range(_NF):
                        v = plsc.load_gather(x_v, [lane + (src + r * _L)])
                        plsc.store_scatter(xt_v, [dst + r * (_L * _TS)], v)
                    return c3

                lax.fori_loop(0, _L, repack, 0)

                def ld(fvec, d):
                    return plsc.load_gather(xt_v, [fvec + d * _TS])

                def fvec(f):
                    return lane + f * (_L * _TS)

                def st(k, v):
                    plsc.store_scatter(o_v, [idx_o + k], v)

                def offdiag(t, c3):
                    bi = ((t >= 4).astype(jnp.int32)
                          + (t >= 7).astype(jnp.int32)
                          + (t >= 9).astype(jnp.int32))
                    bj = t - (4 * bi - (bi * (bi - 1)) // 2) + bi + 1
                    ib = _BS * bi
                    jb = _BS * bj
                    fi = [fvec(ib + a) for a in range(_BS)]
                    fj = [fvec(jb + b) for b in range(_BS)]
                    acc = [[jnp.zeros((_L,), jnp.float32)
                            for _ in range(_BS)] for _ in range(_BS)]
                    for d in range(_D):
                        pa = [ld(fi[a], d) for a in range(_BS)]
                        qa = [ld(fj[b], d) for b in range(_BS)]
                        for a in range(_BS):
                            for b in range(_BS):
                                acc[a][b] = acc[a][b] + pa[a] * qa[b]
                    for a in range(_BS):
                        for b in range(_BS):
                            st(_pair_k(ib + a, jb + b), acc[a][b])
                    return c3

                lax.fori_loop(0, 10, offdiag, 0)

                def diag(t, c3):
                    ib = _BS * t
                    fi = [fvec(ib + a) for a in range(_BS)]
                    acc = {}
                    for a in range(_BS):
                        for b in range(a + 1, _BS):
                            acc[(a, b)] = jnp.zeros((_L,), jnp.float32)
                    for d in range(_D):
                        pa = [ld(fi[a], d) for a in range(_BS)]
                        for a in range(_BS):
                            for b in range(a + 1, _BS):
                                acc[(a, b)] = acc[(a, b)] + pa[a] * pa[b]
                    for a in range(_BS):
                        for b in range(a + 1, _BS):
                            st(_pair_k(ib + a, ib + b), acc[(a, b)])
                    return c3

                lax.fori_loop(0, 5, diag, 0)

                flast = fvec(_NF - 1)
                last = [ld(flast, d) for d in range(_D)]

                def col(t, c3):
                    ib = _BS * t
                    fi = [fvec(ib + a) for a in range(_BS)]
                    acc = [jnp.zeros((_L,), jnp.float32)
                           for _ in range(_BS)]
                    for d in range(_D):
                        pa = [ld(fi[a], d) for a in range(_BS)]
                        for a in range(_BS):
                            acc[a] = acc[a] + pa[a] * last[d]
                    for a in range(_BS):
                        st(_pair_k(ib + a, _NF - 1), acc[a])
                    return c3

                lax.fori_loop(0, 5, col, 0)
                return carry2

            lax.fori_loop(0, n_groups, group_body, 0)
            pltpu.sync_copy(o_v, o_hbm.at[pl.ds(row0 * _NP, _C * _NP)])
            return carry

        lax.fori_loop(0, n_chunks, chunk_body, 0)

    return body


def kernel(inputs):
    n = inputs.shape[0]
    mesh = plsc.VectorSubcoreMesh(core_axis_name="c", subcore_axis_name="s",
                                  num_cores=_NC, num_subcores=_NS)
    f = pl.kernel(
        _make_body(n),
        out_type=jax.ShapeDtypeStruct((n * _NP,), jnp.float32),
        mesh=mesh,
        scratch_types=[pltpu.VMEM((_C * _ROW,), jnp.float32),
                       pltpu.VMEM((_C * _NP,), jnp.float32),
                       pltpu.VMEM((_ROW * _TS,), jnp.float32)],
        compiler_params=pltpu.CompilerParams(use_tc_tiling_on_sc=False,
                                             needs_layout_passes=False),
    )
    flat = f(inputs.reshape(n * _ROW))
    return flat.reshape(n, _NP)


# speedup vs baseline: 8.6286x; 2.7278x over previous
import jax
import jax.numpy as jnp
from jax import lax
from jax.experimental import pallas as pl
from jax.experimental.pallas import tpu as pltpu
from jax.experimental.pallas import tpu_sc as plsc

_NF = 26
_D = 16
_NP = (_NF * (_NF - 1)) // 2
_NPP = 328
_NC = 2
_NS = 16
_NW = _NC * _NS
_L = 16
_BS = 5
_TC = 128
_SEG = 8 * _TC


def _pair_k(i, j):
    return 25 * i - (i * (i - 1)) // 2 + (j - i - 1)


def _make_body(n):
    n_cols = n // _TC
    cols_per = n_cols // _NW
    n_groups = _TC // _L

    def body(x_hbm, o_hbm, x_v, o_v):
        wid = lax.axis_index("s") * _NC + lax.axis_index("c")
        col0 = wid * cols_per

        def col_body(ci, carry):
            tc = col0 + ci
            pltpu.sync_copy(x_hbm.at[:, tc], x_v)

            def group_body(g, carry2):
                b0 = g * _L

                def ld(f, d):
                    return x_v[f * 2 + d // 8, pl.ds((d % 8) * _TC + b0, _L)]

                def st(k, val):
                    o_v[k // 8, pl.ds((k % 8) * _TC + b0, _L)] = val

                def offdiag(t, c3):
                    bi = ((t >= 4).astype(jnp.int32)
                          + (t >= 7).astype(jnp.int32)
                          + (t >= 9).astype(jnp.int32))
                    bj = t - (4 * bi - (bi * (bi - 1)) // 2) + bi + 1
                    ib = _BS * bi
                    jb = _BS * bj
                    acc = [[jnp.zeros((_L,), jnp.float32)
                            for _ in range(_BS)] for _ in range(_BS)]
                    for d in range(_D):
                        pa = [ld(ib + a, d) for a in range(_BS)]
                        qa = [ld(jb + b, d) for b in range(_BS)]
                        for a in range(_BS):
                            for b in range(_BS):
                                acc[a][b] = acc[a][b] + pa[a] * qa[b]
                    for a in range(_BS):
                        for b in range(_BS):
                            st(_pair_k(ib + a, jb + b), acc[a][b])
                    return c3

                lax.fori_loop(0, 10, offdiag, 0)

                def diag(t, c3):
                    ib = _BS * t
                    acc = {}
                    for a in range(_BS):
                        for b in range(a + 1, _BS):
                            acc[(a, b)] = jnp.zeros((_L,), jnp.float32)
                    for d in range(_D):
                        pa = [ld(ib + a, d) for a in range(_BS)]
                        for a in range(_BS):
                            for b in range(a + 1, _BS):
                                acc[(a, b)] = acc[(a, b)] + pa[a] * pa[b]
                    for a in range(_BS):
                        for b in range(a + 1, _BS):
                            st(_pair_k(ib + a, ib + b), acc[(a, b)])
                    return c3

                lax.fori_loop(0, 5, diag, 0)

                last = [ld(_NF - 1, d) for d in range(_D)]

                def col_tiles(t, c3):
                    ib = _BS * t
                    acc = [jnp.zeros((_L,), jnp.float32)
                           for _ in range(_BS)]
                    for d in range(_D):
                        pa = [ld(ib + a, d) for a in range(_BS)]
                        for a in range(_BS):
                            acc[a] = acc[a] + pa[a] * last[d]
                    for a in range(_BS):
                        st(_pair_k(ib + a, _NF - 1), acc[a])
                    return c3

                lax.fori_loop(0, 5, col_tiles, 0)
                return carry2

            lax.fori_loop(0, n_groups, group_body, 0)
            pltpu.sync_copy(o_v, o_hbm.at[:, tc])
            return carry

        lax.fori_loop(0, cols_per, col_body, 0)

    return body


def kernel(inputs):
    n = inputs.shape[0]
    n_cols = n // _TC
    v = inputs.transpose(1, 2, 0)
    v = v.reshape(_NF, 2, 8, n_cols, _TC)
    v = v.transpose(0, 1, 3, 2, 4)
    xv = v.reshape(_NF * 2, n_cols, _SEG)

    mesh = plsc.VectorSubcoreMesh(core_axis_name="c", subcore_axis_name="s",
                                  num_cores=_NC, num_subcores=_NS)
    f = pl.kernel(
        _make_body(n),
        out_type=jax.ShapeDtypeStruct((_NPP // 8, n_cols, _SEG), jnp.float32),
        mesh=mesh,
        scratch_types=[pltpu.VMEM((_NF * 2, _SEG), jnp.float32),
                       pltpu.VMEM((_NPP // 8, _SEG), jnp.float32)],
        compiler_params=pltpu.CompilerParams(use_tc_tiling_on_sc=False,
                                             needs_layout_passes=False),
    )
    out = f(xv)
    y = out.reshape(_NPP // 8, n_cols, 8, _TC)
    y = y.transpose(0, 2, 1, 3).reshape(_NPP, n)
    return y[:_NP].T


# speedup vs baseline: 8.6476x; 1.0022x over previous
import jax
import jax.numpy as jnp
from jax import lax
from jax.experimental import pallas as pl
from jax.experimental.pallas import tpu as pltpu
from jax.experimental.pallas import tpu_sc as plsc

_NF = 26
_NFP = 27
_NB = _NFP // 3
_D = 16
_NP = (_NF * (_NF - 1)) // 2
_NPP = 328
_NC = 2
_NS = 16
_NW = _NC * _NS
_L = 16
_BS = 3
_TC = 128
_SEG = 8 * _TC

_THR = []
_acc = 0
for _i in range(_NB - 1):
    _acc += _NB - 1 - _i
    _THR.append(_acc)
_THR = _THR[:-1] if _THR and _THR[-1] == (_NB * (_NB - 1)) // 2 else _THR


def _pair_k(i, j):
    return 25 * i - (i * (i - 1)) // 2 + (j - i - 1)


def _st_k(i, j):
    return jnp.where(j >= _NF, _NP, _pair_k(i, jnp.minimum(j, _NF - 1)))


def _make_body(n):
    n_cols = n // _TC
    cols_per = n_cols // _NW
    n_groups = _TC // _L
    n_offdiag = (_NB * (_NB - 1)) // 2

    def body(x_hbm, o_hbm, x_v, o_v):
        wid = lax.axis_index("s") * _NC + lax.axis_index("c")
        col0 = wid * cols_per

        def zero_pad(i, carry):
            r = _NF * 2 + i // (_SEG // _L)
            c = (i % (_SEG // _L)) * _L
            x_v[r, pl.ds(c, _L)] = jnp.zeros((_L,), jnp.float32)
            return carry

        lax.fori_loop(0, 2 * (_SEG // _L), zero_pad, 0)

        def col_body(ci, carry):
            tc = col0 + ci
            pltpu.sync_copy(x_hbm.at[:, tc], x_v.at[pl.ds(0, _NF * 2)])

            def group_body(g, carry2):
                b0 = g * _L

                def ld(f, d):
                    return x_v[f * 2 + d // 8, pl.ds((d % 8) * _TC + b0, _L)]

                def st(k, val):
                    o_v[k // 8, pl.ds((k % 8) * _TC + b0, _L)] = val

                def offdiag(t, c3):
                    bi = sum(((t >= thr).astype(jnp.int32) for thr in _THR),
                             jnp.int32(0))
                    bj = t - ((_NB - 1) * bi - (bi * (bi - 1)) // 2) + bi + 1
                    ib = _BS * bi
                    jb = _BS * bj
                    pa = [ld(ib + a, 0) for a in range(_BS)]
                    qa = [ld(jb + b, 0) for b in range(_BS)]
                    acc = [[pa[a] * qa[b] for b in range(_BS)]
                           for a in range(_BS)]
                    for d in range(1, _D):
                        pa = [ld(ib + a, d) for a in range(_BS)]
                        qa = [ld(jb + b, d) for b in range(_BS)]
                        for a in range(_BS):
                            for b in range(_BS):
                                acc[a][b] = acc[a][b] + pa[a] * qa[b]
                    for a in range(_BS):
                        for b in range(_BS):
                            st(_st_k(ib + a, jb + b), acc[a][b])
                    return c3

                lax.fori_loop(0, n_offdiag, offdiag, 0)

                def diag(t, c3):
                    ib = _BS * t
                    pa = [ld(ib + a, 0) for a in range(_BS)]
                    acc = {(a, b): pa[a] * pa[b]
                           for a in range(_BS) for b in range(a + 1, _BS)}
                    for d in range(1, _D):
                        pa = [ld(ib + a, d) for a in range(_BS)]
                        for a in range(_BS):
                            for b in range(a + 1, _BS):
                                acc[(a, b)] = acc[(a, b)] + pa[a] * pa[b]
                    for a in range(_BS):
                        for b in range(a + 1, _BS):
                            st(_st_k(ib + a, ib + b), acc[(a, b)])
                    return c3

                lax.fori_loop(0, _NB, diag, 0)
                return carry2

            lax.fori_loop(0, n_groups, group_body, 0)
            pltpu.sync_copy(o_v, o_hbm.at[:, tc])
            return carry

        lax.fori_loop(0, cols_per, col_body, 0)

    return body


def kernel(inputs):
    n = inputs.shape[0]
    n_cols = n // _TC
    v = inputs.transpose(1, 2, 0)
    v = v.reshape(_NF, 2, 8, n_cols, _TC)
    v = v.transpose(0, 1, 3, 2, 4)
    xv = v.reshape(_NF * 2, n_cols, _SEG)

    mesh = plsc.VectorSubcoreMesh(core_axis_name="c", subcore_axis_name="s",
                                  num_cores=_NC, num_subcores=_NS)
    f = pl.kernel(
        _make_body(n),
        out_type=jax.ShapeDtypeStruct((_NPP // 8, n_cols, _SEG), jnp.float32),
        mesh=mesh,
        scratch_types=[pltpu.VMEM((_NFP * 2, _SEG), jnp.float32),
                       pltpu.VMEM((_NPP // 8, _SEG), jnp.float32)],
        compiler_params=pltpu.CompilerParams(use_tc_tiling_on_sc=False,
                                             needs_layout_passes=False),
    )
    out = f(xv)
    y = out.reshape(_NPP // 8, n_cols, 8, _TC)
    y = y.transpose(0, 2, 1, 3).reshape(_NPP, n)
    return y[:_NP].T
